# Initial kernel scaffold; baseline (speedup 1.0000x reference)
import jax
import jax.numpy as jnp
from jax.experimental import pallas as pl


def kernel(tokens_ids, edge_index, W_embed, W_msg0, W_self0, W_edge0, W_msg1, W_self1, W_edge1):
    raise NotImplementedError("write your pallas kernel here")


# speedup vs baseline: 3.8537x; 3.8537x over previous
import functools

import jax
import jax.numpy as jnp
from jax import lax
from jax.experimental import pallas as pl
from jax.experimental.pallas import tpu as pltpu
from jax.experimental.pallas import tpu_sc as plsc

N = 10000
D = 128
NC, NS = 2, 16
NW = NC * NS
CHUNK = 256
NP = 16384
ACC_N = 10112

_mesh = plsc.VectorSubcoreMesh(core_axis_name="c", subcore_axis_name="s")


def _wid():
    return lax.axis_index("s") * NC + lax.axis_index("c")


def _gather_chunk(table, idx_v, rows_v, sem, n_idx_rows):
    handles = []
    for j in range(n_idx_rows):
        handles.append(
            pltpu.async_copy(table.at[idx_v.at[j]], rows_v.at[pl.ds(j * 128, 128)], sem)
        )
    for h in handles:
        h.wait()


def _emb_body(n0, n1, e0, e1, table, nf_out, ef_out, i0, i1, r0, r1, sem):
    w = _wid()
    n_edge_chunks = ef_out.shape[0] // (NW * CHUNK)

    def do_part(idx0_hbm, idx1_hbm, out_hbm, n_chunks):
        def chunk(i, _):
            base = (w * n_chunks + i) * 2
            pltpu.sync_copy(idx0_hbm.at[pl.ds(base, 2)], i0)
            pltpu.sync_copy(idx1_hbm.at[pl.ds(base, 2)], i1)
            _gather_chunk(table, i0, r0, sem, 2)
            _gather_chunk(table, i1, r1, sem, 2)

            def row(r, _):
                for cc in range(8):
                    sl = pl.ds(cc * 16, 16)
                    r0[r, sl] = (r0[r, sl] + r1[r, sl]) * 0.5
                return 0

            lax.fori_loop(0, CHUNK, row, 0)
            pltpu.sync_copy(r0, out_hbm.at[pl.ds((w * n_chunks + i) * CHUNK, CHUNK)])
            return 0

        lax.fori_loop(0, n_chunks, chunk, 0)

    do_part(n0, n1, nf_out, NP // (NW * CHUNK))
    do_part(e0, e1, ef_out, n_edge_chunks)


def _gather_body(src2d, table, g_out, iv, rv, sem):
    w = _wid()
    n_chunks = g_out.shape[0] // (NW * CHUNK)

    def chunk(i, _):
        base = (w * n_chunks + i) * 2
        pltpu.sync_copy(src2d.at[pl.ds(base, 2)], iv)
        _gather_chunk(table, iv, rv, sem, 2)
        pltpu.sync_copy(rv, g_out.at[pl.ds((w * n_chunks + i) * CHUNK, CHUNK)])
        return 0

    lax.fori_loop(0, n_chunks, chunk, 0)


def _scatter_body(dst2d, msgs, agg_out, iv, rv, zv, acc):
    c = lax.axis_index("c")
    s = lax.axis_index("s")
    w = _wid()
    n_chunks = msgs.shape[0] // (NW * CHUNK)
    per_tile = ACC_N // NS

    def zrow(r, _):
        for cc in range(8):
            zv[r, pl.ds(cc * 16, 16)] = jnp.zeros((16,), jnp.float32)
        return 0

    lax.fori_loop(0, 128, zrow, 0)
    for t in range(per_tile // 128):
        pltpu.sync_copy(zv, acc.at[pl.ds(s * per_tile + t * 128, 128)])
    rem = per_tile % 128
    if rem:
        pltpu.sync_copy(zv.at[pl.ds(0, rem)],
                        acc.at[pl.ds(s * per_tile + (per_tile // 128) * 128, rem)])
    plsc.subcore_barrier()

    def chunk(i, _):
        base = w * n_chunks + i
        pltpu.sync_copy(dst2d.at[pl.ds(base * 2, 2)], iv)
        pltpu.sync_copy(msgs.at[pl.ds(base * CHUNK, CHUNK)], rv)
        for j in range(2):
            pltpu.sync_copy(rv.at[pl.ds(j * 128, 128)], acc.at[iv.at[j]], add=True)
        return 0

    lax.fori_loop(0, n_chunks, chunk, 0)
    plsc.subcore_barrier()
    pltpu.sync_copy(acc.at[pl.ds(s * per_tile, per_tile)],
                    agg_out.at[c, pl.ds(s * per_tile, per_tile)])


def _edge_tc_body(g_ref, ef_ref, wm_ref, we_ref, msgs_ref, efn_ref):
    ef = ef_ref[...]
    m = jnp.maximum(
        jnp.dot(g_ref[...] + ef, wm_ref[...], preferred_element_type=jnp.float32), 0.0)
    msgs_ref[...] = m
    efn_ref[...] = jnp.maximum(
        jnp.dot(ef, we_ref[...], preferred_element_type=jnp.float32) + m, 0.0)


def _node_tc_body(nf_ref, ws_ref, agg_ref, out_ref):
    acc = agg_ref[0] + agg_ref[1]
    out_ref[...] = jnp.maximum(
        jnp.dot(nf_ref[...], ws_ref[...], preferred_element_type=jnp.float32) + acc, 0.0)


def _edge_tc(g, ef, Wm, We):
    EP = g.shape[0]
    BE = 2048
    grid = (EP // BE,)
    return pl.pallas_call(
        _edge_tc_body,
        grid=grid,
        in_specs=[
            pl.BlockSpec((BE, D), lambda i: (i, 0)),
            pl.BlockSpec((BE, D), lambda i: (i, 0)),
            pl.BlockSpec((D, D), lambda i: (0, 0)),
            pl.BlockSpec((D, D), lambda i: (0, 0)),
        ],
        out_specs=[
            pl.BlockSpec((BE, D), lambda i: (i, 0)),
            pl.BlockSpec((BE, D), lambda i: (i, 0)),
        ],
        out_shape=[
            jax.ShapeDtypeStruct((EP, D), jnp.float32),
            jax.ShapeDtypeStruct((EP, D), jnp.float32),
        ],
        compiler_params=pltpu.CompilerParams(dimension_semantics=("parallel",)),
    )(g, ef, Wm, We)


def _node_tc(nf_pad, Ws, agg):
    BN = 2000
    return pl.pallas_call(
        _node_tc_body,
        grid=(N // BN,),
        in_specs=[
            pl.BlockSpec((BN, D), lambda i: (i, 0)),
            pl.BlockSpec((D, D), lambda i: (0, 0)),
            pl.BlockSpec((2, BN, D), lambda i: (0, i, 0)),
        ],
        out_specs=pl.BlockSpec((BN, D), lambda i: (i, 0)),
        out_shape=jax.ShapeDtypeStruct((N, D), jnp.float32),
        compiler_params=pltpu.CompilerParams(dimension_semantics=("parallel",)),
    )(nf_pad, Ws, agg)


def _sc_emb(n0, n1, e0, e1, W_embed, EP):
    call = pl.kernel(
        _emb_body,
        out_type=[
            jax.ShapeDtypeStruct((NP, D), jnp.float32),
            jax.ShapeDtypeStruct((EP, D), jnp.float32),
        ],
        mesh=_mesh,
        scratch_types=[
            pltpu.VMEM((2, 128), jnp.int32),
            pltpu.VMEM((2, 128), jnp.int32),
            pltpu.VMEM((CHUNK, D), jnp.float32),
            pltpu.VMEM((CHUNK, D), jnp.float32),
            pltpu.SemaphoreType.DMA,
        ],
    )
    return call(n0, n1, e0, e1, W_embed)


def _sc_gather(src2d, table, EP):
    call = pl.kernel(
        _gather_body,
        out_type=jax.ShapeDtypeStruct((EP, D), jnp.float32),
        mesh=_mesh,
        scratch_types=[
            pltpu.VMEM((2, 128), jnp.int32),
            pltpu.VMEM((CHUNK, D), jnp.float32),
            pltpu.SemaphoreType.DMA,
        ],
    )
    return call(src2d, table)


def _sc_scatter(dst2d, msgs):
    call = pl.kernel(
        _scatter_body,
        out_type=jax.ShapeDtypeStruct((NC, ACC_N, D), jnp.float32),
        mesh=_mesh,
        scratch_types=[
            pltpu.VMEM((2, 128), jnp.int32),
            pltpu.VMEM((CHUNK, D), jnp.float32),
            pltpu.VMEM((128, D), jnp.float32),
            pltpu.VMEM_SHARED((ACC_N, D), jnp.float32),
        ],
    )
    return call(dst2d, msgs)


def kernel(tokens_ids, edge_index, W_embed, W_msg0, W_self0, W_edge0,
           W_msg1, W_self1, W_edge1):
    T = tokens_ids.shape[0]
    E = edge_index.shape[1]
    V = W_embed.shape[0]
    EP = ((E + NW * CHUNK - 1) // (NW * CHUNK)) * (NW * CHUNK)

    t0 = tokens_ids[:, 0].astype(jnp.int32)
    t1 = tokens_ids[:, 1].astype(jnp.int32)
    pad_n = (jnp.arange(NP - N, dtype=jnp.int32) * 37) % V
    pad_e = (jnp.arange(EP - E, dtype=jnp.int32) * 37) % V
    n0 = jnp.concatenate([t0[:N], pad_n]).reshape(NP // 128, 128)
    n1 = jnp.concatenate([t1[:N], pad_n]).reshape(NP // 128, 128)
    e0 = jnp.concatenate([t0[N:], pad_e]).reshape(EP // 128, 128)
    e1 = jnp.concatenate([t1[N:], pad_e]).reshape(EP // 128, 128)

    src = edge_index[0].astype(jnp.int32)
    dst = edge_index[1].astype(jnp.int32)
    pad_src = jnp.arange(EP - E, dtype=jnp.int32) % N
    pad_dst = N + (jnp.arange(EP - E, dtype=jnp.int32) % (ACC_N - N))
    src2d = jnp.concatenate([src, pad_src]).reshape(EP // 128, 128)
    dst2d = jnp.concatenate([dst, pad_dst]).reshape(EP // 128, 128)

    nf_pad, ef = _sc_emb(n0, n1, e0, e1, W_embed, EP)

    table = nf_pad
    for (Wm, Ws, We) in ((W_msg0, W_self0, W_edge0), (W_msg1, W_self1, W_edge1)):
        g = _sc_gather(src2d, table, EP)
        msgs, ef = _edge_tc(g, ef, Wm, We)
        agg = _sc_scatter(dst2d, msgs)
        table = _node_tc(table, Ws, agg)

    return jnp.concatenate([table, ef[:E]], axis=0)


# speedup vs baseline: 4.3917x; 1.1396x over previous
import jax
import jax.numpy as jnp
from jax import lax
from jax.experimental import pallas as pl
from jax.experimental.pallas import tpu as pltpu
from jax.experimental.pallas import tpu_sc as plsc

N = 10000
D = 128
NC, NS = 2, 16
NW = NC * NS
CHUNK = 256
GCHUNK = 512
SCHUNK = 256
NP = 16384
ACC_N = 10112

_mesh = plsc.VectorSubcoreMesh(core_axis_name="c", subcore_axis_name="s")


def _wid():
    return lax.axis_index("s") * NC + lax.axis_index("c")


def _gather_chunk(table, idx_v, rows_v, sem, n_idx_rows):
    handles = []
    for j in range(n_idx_rows):
        handles.append(
            pltpu.async_copy(table.at[idx_v.at[j]], rows_v.at[pl.ds(j * 128, 128)], sem)
        )
    for h in handles:
        h.wait()


def _emb_body(n0, n1, e0, e1, table, nf_out, ef_out, i0, i1, r0, r1, sem):
    w = _wid()
    n_edge_chunks = ef_out.shape[0] // (NW * CHUNK)

    def do_part(idx0_hbm, idx1_hbm, out_hbm, n_chunks):
        def chunk(i, _):
            base = (w * n_chunks + i) * 2
            pltpu.sync_copy(idx0_hbm.at[pl.ds(base, 2)], i0)
            pltpu.sync_copy(idx1_hbm.at[pl.ds(base, 2)], i1)
            _gather_chunk(table, i0, r0, sem, 2)
            _gather_chunk(table, i1, r1, sem, 2)

            def row(r, _):
                for cc in range(8):
                    sl = pl.ds(cc * 16, 16)
                    r0[r, sl] = (r0[r, sl] + r1[r, sl]) * 0.5
                return 0

            lax.fori_loop(0, CHUNK, row, 0)
            pltpu.sync_copy(r0, out_hbm.at[pl.ds((w * n_chunks + i) * CHUNK, CHUNK)])
            return 0

        lax.fori_loop(0, n_chunks, chunk, 0)

    do_part(n0, n1, nf_out, NP // (NW * CHUNK))
    do_part(e0, e1, ef_out, n_edge_chunks)


def _gather_body(src2d, table, g_out, iv, rv, sem):
    w = _wid()
    n_chunks = g_out.shape[0] // (NW * GCHUNK)
    k = GCHUNK // 128

    def chunk(i, _):
        base = (w * n_chunks + i) * k
        pltpu.sync_copy(src2d.at[pl.ds(base, k)], iv)
        _gather_chunk(table, iv, rv, sem, k)
        pltpu.sync_copy(rv, g_out.at[pl.ds((w * n_chunks + i) * GCHUNK, GCHUNK)])
        return 0

    lax.fori_loop(0, n_chunks, chunk, 0)


def _scatter_body(dst2d, msgs, agg_out, iv, rv, acc):
    c = lax.axis_index("c")
    s = lax.axis_index("s")
    w = _wid()
    n_chunks = msgs.shape[0] // (NW * SCHUNK)
    k = SCHUNK // 128
    per_tile = ACC_N // NS

    def zrow(r, _):
        for cc in range(8):
            rv[r, pl.ds(cc * 16, 16)] = jnp.zeros((16,), jnp.float32)
        return 0

    lax.fori_loop(0, 128, zrow, 0)
    for t in range(per_tile // 128):
        pltpu.sync_copy(rv.at[pl.ds(0, 128)], acc.at[pl.ds(s * per_tile + t * 128, 128)])
    rem = per_tile % 128
    if rem:
        pltpu.sync_copy(rv.at[pl.ds(0, rem)],
                        acc.at[pl.ds(s * per_tile + (per_tile // 128) * 128, rem)])
    plsc.subcore_barrier()

    def chunk(i, _):
        base = w * n_chunks + i
        pltpu.sync_copy(dst2d.at[pl.ds(base * k, k)], iv)
        pltpu.sync_copy(msgs.at[pl.ds(base * SCHUNK, SCHUNK)], rv)
        for j in range(k):
            pltpu.sync_copy(rv.at[pl.ds(j * 128, 128)], acc.at[iv.at[j]], add=True)
        return 0

    lax.fori_loop(0, n_chunks, chunk, 0)
    plsc.subcore_barrier()
    pltpu.sync_copy(acc.at[pl.ds(s * per_tile, per_tile)],
                    agg_out.at[c, pl.ds(s * per_tile, per_tile)])


def _edge_tc_body(g_ref, ef_ref, wm_ref, we_ref, msgs_ref, efn_ref):
    ef = ef_ref[...]
    m = jnp.maximum(
        jnp.dot(g_ref[...] + ef, wm_ref[...], preferred_element_type=jnp.float32), 0.0)
    msgs_ref[...] = m
    efn_ref[...] = jnp.maximum(
        jnp.dot(ef, we_ref[...], preferred_element_type=jnp.float32) + m, 0.0)


def _node_tc_body(nf_ref, ws_ref, agg_ref, out_ref):
    acc = agg_ref[0] + agg_ref[1]
    out_ref[...] = jnp.maximum(
        jnp.dot(nf_ref[...], ws_ref[...], preferred_element_type=jnp.float32) + acc, 0.0)


def _node_final_body(_aliased_ref, nf_ref, ws_ref, agg_ref, out_ref):
    acc = agg_ref[0] + agg_ref[1]
    out_ref[...] = jnp.maximum(
        jnp.dot(nf_ref[...], ws_ref[...], preferred_element_type=jnp.float32) + acc, 0.0)


BE = 2000


def _edge_tc(g, ef, Wm, We, E):
    EP = g.shape[0]
    return pl.pallas_call(
        _edge_tc_body,
        grid=(E // BE,),
        in_specs=[
            pl.BlockSpec((BE, D), lambda i: (i, 0)),
            pl.BlockSpec((BE, D), lambda i: (i, 0)),
            pl.BlockSpec((D, D), lambda i: (0, 0)),
            pl.BlockSpec((D, D), lambda i: (0, 0)),
        ],
        out_specs=[
            pl.BlockSpec((BE, D), lambda i: (i, 0)),
            pl.BlockSpec((BE, D), lambda i: (i, 0)),
        ],
        out_shape=[
            jax.ShapeDtypeStruct((EP, D), jnp.float32),
            jax.ShapeDtypeStruct((E, D), jnp.float32),
        ],
        compiler_params=pltpu.CompilerParams(dimension_semantics=("parallel",)),
    )(g, ef, Wm, We)


def _edge_tc_final(g, ef, Wm, We, E, T):
    EP = g.shape[0]
    nblk = N // BE
    return pl.pallas_call(
        _edge_tc_body,
        grid=(E // BE,),
        in_specs=[
            pl.BlockSpec((BE, D), lambda i: (i, 0)),
            pl.BlockSpec((BE, D), lambda i: (i, 0)),
            pl.BlockSpec((D, D), lambda i: (0, 0)),
            pl.BlockSpec((D, D), lambda i: (0, 0)),
        ],
        out_specs=[
            pl.BlockSpec((BE, D), lambda i: (i, 0)),
            pl.BlockSpec((BE, D), lambda i: (i + nblk, 0)),
        ],
        out_shape=[
            jax.ShapeDtypeStruct((EP, D), jnp.float32),
            jax.ShapeDtypeStruct((T, D), jnp.float32),
        ],
        compiler_params=pltpu.CompilerParams(dimension_semantics=("parallel",)),
    )(g, ef, Wm, We)


def _node_tc(nf_pad, Ws, agg):
    BN = 2000
    return pl.pallas_call(
        _node_tc_body,
        grid=(N // BN,),
        in_specs=[
            pl.BlockSpec((BN, D), lambda i: (i, 0)),
            pl.BlockSpec((D, D), lambda i: (0, 0)),
            pl.BlockSpec((2, BN, D), lambda i: (0, i, 0)),
        ],
        out_specs=pl.BlockSpec((BN, D), lambda i: (i, 0)),
        out_shape=jax.ShapeDtypeStruct((N, D), jnp.float32),
        compiler_params=pltpu.CompilerParams(dimension_semantics=("parallel",)),
    )(nf_pad, Ws, agg)


def _node_tc_final(allfeats, nf, Ws, agg):
    BN = 2000
    T = allfeats.shape[0]
    return pl.pallas_call(
        _node_final_body,
        grid=(N // BN,),
        in_specs=[
            pl.BlockSpec((BN, D), lambda i: (i, 0)),
            pl.BlockSpec((BN, D), lambda i: (i, 0)),
            pl.BlockSpec((D, D), lambda i: (0, 0)),
            pl.BlockSpec((2, BN, D), lambda i: (0, i, 0)),
        ],
        out_specs=pl.BlockSpec((BN, D), lambda i: (i, 0)),
        out_shape=jax.ShapeDtypeStruct((T, D), jnp.float32),
        input_output_aliases={0: 0},
        compiler_params=pltpu.CompilerParams(dimension_semantics=("parallel",)),
    )(allfeats, nf, Ws, agg)


def _sc_emb(n0, n1, e0, e1, W_embed, EP):
    call = pl.kernel(
        _emb_body,
        out_type=[
            jax.ShapeDtypeStruct((NP, D), jnp.float32),
            jax.ShapeDtypeStruct((EP, D), jnp.float32),
        ],
        mesh=_mesh,
        scratch_types=[
            pltpu.VMEM((2, 128), jnp.int32),
            pltpu.VMEM((2, 128), jnp.int32),
            pltpu.VMEM((CHUNK, D), jnp.float32),
            pltpu.VMEM((CHUNK, D), jnp.float32),
            pltpu.SemaphoreType.DMA,
        ],
    )
    return call(n0, n1, e0, e1, W_embed)


def _sc_gather(src2d, table, EP):
    call = pl.kernel(
        _gather_body,
        out_type=jax.ShapeDtypeStruct((EP, D), jnp.float32),
        mesh=_mesh,
        scratch_types=[
            pltpu.VMEM((GCHUNK // 128, 128), jnp.int32),
            pltpu.VMEM((GCHUNK, D), jnp.float32),
            pltpu.SemaphoreType.DMA,
        ],
    )
    return call(src2d, table)


def _sc_scatter(dst2d, msgs):
    call = pl.kernel(
        _scatter_body,
        out_type=jax.ShapeDtypeStruct((NC, ACC_N, D), jnp.float32),
        mesh=_mesh,
        scratch_types=[
            pltpu.VMEM((SCHUNK // 128, 128), jnp.int32),
            pltpu.VMEM((SCHUNK, D), jnp.float32),
            pltpu.VMEM_SHARED((ACC_N, D), jnp.float32),
        ],
    )
    return call(dst2d, msgs)


def kernel(tokens_ids, edge_index, W_embed, W_msg0, W_self0, W_edge0,
           W_msg1, W_self1, W_edge1):
    T = tokens_ids.shape[0]
    E = edge_index.shape[1]
    V = W_embed.shape[0]
    EP = ((E + NW * GCHUNK - 1) // (NW * GCHUNK)) * (NW * GCHUNK)

    t0 = tokens_ids[:, 0].astype(jnp.int32)
    t1 = tokens_ids[:, 1].astype(jnp.int32)
    pad_n = (jnp.arange(NP - N, dtype=jnp.int32) * 37) % V
    pad_e = (jnp.arange(EP - E, dtype=jnp.int32) * 37) % V
    n0 = jnp.concatenate([t0[:N], pad_n]).reshape(NP // 128, 128)
    n1 = jnp.concatenate([t1[:N], pad_n]).reshape(NP // 128, 128)
    e0 = jnp.concatenate([t0[N:], pad_e]).reshape(EP // 128, 128)
    e1 = jnp.concatenate([t1[N:], pad_e]).reshape(EP // 128, 128)

    src = edge_index[0].astype(jnp.int32)
    dst = edge_index[1].astype(jnp.int32)
    pad_src = jnp.arange(EP - E, dtype=jnp.int32) % N
    pad_dst = N + (jnp.arange(EP - E, dtype=jnp.int32) % (ACC_N - N))
    src2d = jnp.concatenate([src, pad_src]).reshape(EP // 128, 128)
    dst2d = jnp.concatenate([dst, pad_dst]).reshape(EP // 128, 128)

    nf_pad, ef = _sc_emb(n0, n1, e0, e1, W_embed, EP)

    g = _sc_gather(src2d, nf_pad, EP)
    msgs, ef = _edge_tc(g, ef, W_msg0, W_edge0, E)
    agg = _sc_scatter(dst2d, msgs)
    nf = _node_tc(nf_pad, W_self0, agg)

    g = _sc_gather(src2d, nf, EP)
    msgs, allfeats = _edge_tc_final(g, ef, W_msg1, W_edge1, E, T)
    agg = _sc_scatter(dst2d, msgs)
    return _node_tc_final(allfeats, nf, W_self1, agg)
